# baseline (speedup 1.0000x reference)
import functools

import jax
import jax.numpy as jnp
from jax import lax
from jax.experimental import pallas as pl
from jax.experimental.pallas import tpu as pltpu
from jax.experimental.pallas import tpu_sc as plsc

_ALPHA = 4
_IDX = (0, 4, 8, 12, 16, 21, 25, 29, 33, 37, 42, 46, 50, 54, 58, 63)
_C, _T, _H, _W = 3, 64, 224, 224
_TS = _T // _ALPHA
_HHALF = _H // 2
_NPIECE = _C * _TS * 2
_NWORKER = 32
_PER_W = _NPIECE // _NWORKER


def _tc_body(x_ref, fast_ref, sem):
    c = pl.program_id(0)
    cp = pltpu.make_async_copy(x_ref.at[0], fast_ref.at[c], sem)
    cp.start()
    cp.wait()


def _fast_copy(frames):
    return pl.pallas_call(
        _tc_body,
        grid=(_C,),
        in_specs=[pl.BlockSpec((1, _T, _H, _W), lambda c: (c, 0, 0, 0))],
        out_specs=pl.BlockSpec(memory_space=pl.ANY),
        out_shape=jax.ShapeDtypeStruct((_C, _T, _H, _W), frames.dtype),
        scratch_shapes=[pltpu.SemaphoreType.DMA],
    )(frames)


def _sc_body(x_hbm, slow_hbm, buf):
    wid = lax.axis_index("s") * 2 + lax.axis_index("c")
    for i in range(_PER_W):
        p = wid * _PER_W + i
        c = p // (_TS * 2)
        r = p % (_TS * 2)
        j = r // 2
        h = r % 2
        g = (63 * j) // 15
        pltpu.sync_copy(x_hbm.at[c, g, pl.ds(h * _HHALF, _HHALF)], buf)
        pltpu.sync_copy(buf, slow_hbm.at[c, j, pl.ds(h * _HHALF, _HHALF)])


def _slow_gather(frames):
    mesh = plsc.VectorSubcoreMesh(core_axis_name="c", subcore_axis_name="s")
    k = functools.partial(
        pl.kernel,
        mesh=mesh,
        out_type=jax.ShapeDtypeStruct((_C, _TS, _H, _W), frames.dtype),
        scratch_types=[pltpu.VMEM((_HHALF, _W), jnp.float32)],
    )(_sc_body)
    return k(frames)


def kernel(frames):
    slow = _slow_gather(frames)
    fast = _fast_copy(frames)
    return (slow, fast)


# speedup vs baseline: 1.6723x; 1.6723x over previous
import jax
import jax.numpy as jnp
from jax.experimental import pallas as pl
from jax.experimental.pallas import tpu as pltpu

_ALPHA = 4
_IDX = (0, 4, 8, 12, 16, 21, 25, 29, 33, 37, 42, 46, 50, 54, 58, 63)
_C, _T, _H, _W = 3, 64, 224, 224
_TS = _T // _ALPHA
_CH = 8
_WPC = _T // _CH
_NCH = _C * _WPC
_SELC = _TS // _WPC
_K = 8
_D = 4


def _body(x_ref, slow_ref, fast_ref, *scratch):
    bufs = scratch[:_K]
    sin = scratch[_K:2 * _K]
    sout = scratch[2 * _K:3 * _K]

    def in_cp(m):
        c, w = divmod(m, _WPC)
        return pltpu.make_async_copy(
            x_ref.at[c, pl.ds(w * _CH, _CH)], bufs[m % _K], sin[m % _K]
        )

    def out_cps(m):
        c, w = divmod(m, _WPC)
        cps = [
            pltpu.make_async_copy(
                bufs[m % _K], fast_ref.at[c, pl.ds(w * _CH, _CH)], sout[m % _K]
            )
        ]
        for k in range(_SELC):
            j = w * _SELC + k
            g = _IDX[j] - w * _CH
            cps.append(
                pltpu.make_async_copy(
                    bufs[m % _K].at[g], slow_ref.at[c, j], sout[m % _K]
                )
            )
        return cps

    pending = {}
    for m in range(_D):
        in_cp(m).start()
    for m in range(_NCH):
        in_cp(m).wait()
        cps = out_cps(m)
        for cp in cps:
            cp.start()
        pending[m] = cps
        nm = m + _D
        if nm < _NCH:
            prev = nm - _K
            if prev >= 0:
                for cp in pending.pop(prev):
                    cp.wait()
            in_cp(nm).start()
    for m in sorted(pending):
        for cp in pending[m]:
            cp.wait()


def kernel(frames):
    C, T, H, W = frames.shape
    Ts = T // _ALPHA
    slow, fast = pl.pallas_call(
        _body,
        in_specs=[pl.BlockSpec(memory_space=pl.ANY)],
        out_specs=[
            pl.BlockSpec(memory_space=pl.ANY),
            pl.BlockSpec(memory_space=pl.ANY),
        ],
        out_shape=[
            jax.ShapeDtypeStruct((C, Ts, H, W), frames.dtype),
            jax.ShapeDtypeStruct((C, T, H, W), frames.dtype),
        ],
        scratch_shapes=(
            [pltpu.VMEM((_CH, _H, _W), jnp.float32)] * _K
            + [pltpu.SemaphoreType.DMA] * (2 * _K)
        ),
    )(frames)
    return (slow, fast)


# speedup vs baseline: 1.6972x; 1.0149x over previous
import jax
import jax.numpy as jnp
from jax.experimental import pallas as pl
from jax.experimental.pallas import tpu as pltpu

_ALPHA = 4
_IDX = (0, 4, 8, 12, 16, 21, 25, 29, 33, 37, 42, 46, 50, 54, 58, 63)
_C, _T, _H, _W = 3, 64, 224, 224
_TS = _T // _ALPHA
_CH = 8
_WPC = _T // _CH
_NCH = _C * _WPC
_SELC = _TS // _WPC
_K = 12
_D = 6


def _body(x_ref, slow_ref, fast_ref, *scratch):
    bufs = scratch[:_K]
    sin = scratch[_K:2 * _K]
    sout = scratch[2 * _K:3 * _K]

    def in_cp(m):
        c, w = divmod(m, _WPC)
        return pltpu.make_async_copy(
            x_ref.at[c, pl.ds(w * _CH, _CH)], bufs[m % _K], sin[m % _K]
        )

    def out_cps(m):
        c, w = divmod(m, _WPC)
        cps = [
            pltpu.make_async_copy(
                bufs[m % _K], fast_ref.at[c, pl.ds(w * _CH, _CH)], sout[m % _K]
            )
        ]
        for k in range(_SELC):
            j = w * _SELC + k
            g = _IDX[j] - w * _CH
            cps.append(
                pltpu.make_async_copy(
                    bufs[m % _K].at[g], slow_ref.at[c, j], sout[m % _K]
                )
            )
        return cps

    pending = {}
    for m in range(_D):
        in_cp(m).start()
    for m in range(_NCH):
        in_cp(m).wait()
        cps = out_cps(m)
        for cp in cps:
            cp.start()
        pending[m] = cps
        nm = m + _D
        if nm < _NCH:
            prev = nm - _K
            if prev >= 0:
                for cp in pending.pop(prev):
                    cp.wait()
            in_cp(nm).start()
    for m in sorted(pending):
        for cp in pending[m]:
            cp.wait()


def kernel(frames):
    C, T, H, W = frames.shape
    Ts = T // _ALPHA
    slow, fast = pl.pallas_call(
        _body,
        in_specs=[pl.BlockSpec(memory_space=pl.ANY)],
        out_specs=[
            pl.BlockSpec(memory_space=pl.ANY),
            pl.BlockSpec(memory_space=pl.ANY),
        ],
        out_shape=[
            jax.ShapeDtypeStruct((C, Ts, H, W), frames.dtype),
            jax.ShapeDtypeStruct((C, T, H, W), frames.dtype),
        ],
        scratch_shapes=(
            [pltpu.VMEM((_CH, _H, _W), jnp.float32)] * _K
            + [pltpu.SemaphoreType.DMA] * (2 * _K)
        ),
    )(frames)
    return (slow, fast)
